# baseline (speedup 1.0000x reference)
import functools
import jax
import jax.numpy as jnp
import numpy as np
from jax import lax
from jax.experimental import pallas as pl
from jax.experimental.pallas import tpu as pltpu
from jax.experimental.pallas import tpu_sc as plsc

_N = 100000
_EMB = 64
_NCAT = 9
_NF = 41
_PK = 128
_TOT = 174
_TROWS = 1203
_OA, _OB, _OC = 119, 839, 1199

_NW = 32
_RPW = 3136
_C = 96
_CT = 64
_NFULL = 32
_BLK = 3136


def _build_body(e0, e1, e2, e3, e4, e5, e6, e7, e8, w1_ref, t_ref):
    embcat = jnp.concatenate(
        [e0[...], e1[...], e2[...], e3[...], e4[...], e5[...], e6[...],
         e7[...], e8[...]], axis=0)
    f = jnp.dot(embcat, w1_ref[...], preferred_element_type=jnp.float32)
    rf = lax.broadcasted_iota(jnp.int32, (_TROWS, _TOT), 0).astype(jnp.float32)
    cf = lax.broadcasted_iota(jnp.int32, (_TROWS, _TOT), 1).astype(jnp.float32)

    def fdiv(v, d):
        return jnp.floor((v + 0.5) * (1.0 / d))

    m = (rf < _OA) & (cf == rf)

    ra = rf - _OA
    i1 = fdiv(ra, 144.0)
    rem = ra - i1 * 144.0
    i2 = fdiv(rem, 12.0)
    i3 = rem - i2 * 12.0
    in_a = (rf >= _OA) & (rf < _OB)
    m = m | (in_a & ((cf == 119.0 + i1) | (cf == 124.0 + i2)
                     | (cf == 136.0 + i3)))

    rb = rf - _OB
    j1 = fdiv(rb, 36.0)
    remb = rb - j1 * 36.0
    j2 = fdiv(remb, 6.0)
    j3 = remb - j2 * 6.0
    in_b = (rf >= _OB) & (rf < _OC)
    m = m | (in_b & ((cf == 148.0 + j1) | (cf == 158.0 + j2)
                     | (cf == 164.0 + j3)))

    rc = rf - _OC
    k1 = fdiv(rc, 2.0)
    k2 = rc - k1 * 2.0
    in_c = (rf >= _OC) & (rf < _OC + 4)
    m = m | (in_c & ((cf == 170.0 + k1) | (cf == 172.0 + k2)))

    sel = m.astype(jnp.float32)
    t_ref[...] = jnp.dot(sel, f, preferred_element_type=jnp.float32)


def _prep_body(x_ref, wcomb_ref, add_ref, pk_ref):
    pk_ref[...] = (jnp.dot(x_ref[...], wcomb_ref[...],
                           preferred_element_type=jnp.float32)
                   + add_ref[...])


def _sc_body(t_hbm, pk_hbm, out_hbm, t_v, pk0_v, pk1_v, ao0_v, ao1_v, sems):
    wid = lax.axis_index("s") * 2 + lax.axis_index("c")
    row0 = wid * _RPW
    is_last = wid == _NW - 1
    nfull = jnp.where(is_last, 29, _NFULL)
    pltpu.sync_copy(t_hbm, t_v)
    bufs = [(pk0_v, ao0_v), (pk1_v, ao1_v)]

    def start_in(r0, b, rows):
        pltpu.async_copy(pk_hbm.at[pl.ds(r0, rows)],
                         bufs[b][0].at[pl.ds(0, rows)], sems.at[b])

    def wait_in(b, rows):
        pltpu.make_async_copy(pk_hbm.at[pl.ds(0, rows)],
                              bufs[b][0].at[pl.ds(0, rows)],
                              sems.at[b]).wait()

    def start_out(r0, b, rows):
        pltpu.async_copy(bufs[b][1].at[pl.ds(0, rows)],
                         out_hbm.at[pl.ds(r0, rows)], sems.at[2 + b])

    def wait_out(b, rows):
        pltpu.make_async_copy(bufs[b][1].at[pl.ds(0, rows)],
                              out_hbm.at[pl.ds(0, rows)],
                              sems.at[2 + b]).wait()

    def compute(b, rows):
        pk_b, ao_b = bufs[b]

        def row_body(r):
            ivec = pk_b[r, pl.ds(_EMB, 16)].astype(jnp.int32)
            i0 = ivec[0]
            i1 = ivec[1]
            i2 = ivec[2]
            i3 = ivec[3]
            for k in range(4):
                o = k * 16
                ao_b[r, pl.ds(o, 16)] = (
                    pk_b[r, pl.ds(o, 16)]
                    + t_v[pl.ds(i0 + o, 16)]
                    + t_v[pl.ds(i1 + o, 16)]
                    + t_v[pl.ds(i2 + o, 16)]
                    + t_v[pl.ds(i3 + o, 16)])

        plsc.parallel_loop(0, rows, 1, unroll=8)(row_body)

    start_in(row0, 0, _C)
    start_in(row0 + _C, 1, _C)

    def chunk_body(ci, carry):
        def do(b):
            wait_in(b, _C)
            compute(b, _C)
            r0 = row0 + ci * _C
            start_out(r0, b, _C)

            @pl.when(ci + 2 < nfull)
            def _():
                wait_out(b, _C)
                start_in(row0 + (ci + 2) * _C, b, _C)

        @pl.when(ci % 2 == 0)
        def _():
            do(0)

        @pl.when(ci % 2 == 1)
        def _():
            do(1)

        return carry

    lax.fori_loop(0, nfull, chunk_body, 0)
    wait_out(0, _C)
    wait_out(1, _C)

    @pl.when(jnp.logical_not(is_last))
    def _():
        r0 = row0 + _NFULL * _C
        start_in(r0, 0, _CT)
        wait_in(0, _CT)
        compute(0, _CT)
        start_out(r0, 0, _CT)
        wait_out(0, _CT)


def kernel(x, emb0, emb1, emb2, emb3, emb4, emb5, emb6, emb7, emb8, W, b):
    n = x.shape[0]
    w1 = W[:_EMB]

    t = pl.pallas_call(
        _build_body,
        out_shape=jax.ShapeDtypeStruct((_TROWS, _EMB), jnp.float32),
    )(emb0, emb1, emb2, emb3, emb4, emb5, emb6, emb7, emb8, w1)

    wc_np = np.zeros((_NF, _PK), np.float32)
    wc_np[0, _EMB + 0] = 64.0
    wc_np[1, _EMB + 1], wc_np[2, _EMB + 1], wc_np[3, _EMB + 1] = (
        144.0 * 64, 12.0 * 64, 64.0)
    wc_np[4, _EMB + 2], wc_np[5, _EMB + 2], wc_np[6, _EMB + 2] = (
        36.0 * 64, 6.0 * 64, 64.0)
    wc_np[7, _EMB + 3], wc_np[8, _EMB + 3] = 2.0 * 64, 64.0
    wcomb = jnp.asarray(wc_np).at[_NCAT:, :_EMB].set(W[_EMB:])
    add_np = np.zeros((1, _PK), np.float32)
    add_np[0, _EMB:_EMB + 4] = [0.5, _OA * 64 + 0.5, _OB * 64 + 0.5,
                                _OC * 64 + 0.5]
    add = jnp.asarray(add_np).at[0, :_EMB].set(b)

    nb = (n + _BLK - 1) // _BLK
    pk = pl.pallas_call(
        _prep_body,
        grid=(nb,),
        in_specs=[
            pl.BlockSpec((_BLK, _NF), lambda i: (i, 0)),
            pl.BlockSpec((_NF, _PK), lambda i: (0, 0)),
            pl.BlockSpec((1, _PK), lambda i: (0, 0)),
        ],
        out_specs=pl.BlockSpec((_BLK, _PK), lambda i: (i, 0)),
        out_shape=jax.ShapeDtypeStruct((n, _PK), jnp.float32),
    )(x, wcomb, add)

    sc_fn = functools.partial(
        pl.kernel,
        mesh=plsc.VectorSubcoreMesh(core_axis_name="c", subcore_axis_name="s"),
        out_type=jax.ShapeDtypeStruct((n, _EMB), jnp.float32),
        scratch_types=[
            pltpu.VMEM((_TROWS * _EMB,), jnp.float32),
            pltpu.VMEM((_C, _PK), jnp.float32),
            pltpu.VMEM((_C, _PK), jnp.float32),
            pltpu.VMEM((_C, _EMB), jnp.float32),
            pltpu.VMEM((_C, _EMB), jnp.float32),
            pltpu.SemaphoreType.DMA((4,)),
        ],
    )(_sc_body)
    return sc_fn(t.reshape(-1), pk)


# speedup vs baseline: 1.0592x; 1.0592x over previous
import functools
import jax
import jax.numpy as jnp
import numpy as np
from jax import lax
from jax.experimental import pallas as pl
from jax.experimental.pallas import tpu as pltpu
from jax.experimental.pallas import tpu_sc as plsc

_N = 100000
_EMB = 64
_NCAT = 9
_NF = 41
_PK = 128
_TOT = 174
_TROWS = 1203
_OA, _OB, _OC = 119, 839, 1199

_NW = 32
_RPW = 3136
_C = 96
_CT = 64
_NFULL = 32
_BLK = 3136


def _build_body(e0, e1, e2, e3, e4, e5, e6, e7, e8, w1_ref, t_ref):
    embcat = jnp.concatenate(
        [e0[...], e1[...], e2[...], e3[...], e4[...], e5[...], e6[...],
         e7[...], e8[...]], axis=0)
    f = jnp.dot(embcat, w1_ref[...], preferred_element_type=jnp.float32)
    rf = lax.broadcasted_iota(jnp.int32, (_TROWS, _TOT), 0).astype(jnp.float32)
    cf = lax.broadcasted_iota(jnp.int32, (_TROWS, _TOT), 1).astype(jnp.float32)

    def fdiv(v, d):
        return jnp.floor((v + 0.5) * (1.0 / d))

    m = (rf < _OA) & (cf == rf)

    ra = rf - _OA
    i1 = fdiv(ra, 144.0)
    rem = ra - i1 * 144.0
    i2 = fdiv(rem, 12.0)
    i3 = rem - i2 * 12.0
    in_a = (rf >= _OA) & (rf < _OB)
    m = m | (in_a & ((cf == 119.0 + i1) | (cf == 124.0 + i2)
                     | (cf == 136.0 + i3)))

    rb = rf - _OB
    j1 = fdiv(rb, 36.0)
    remb = rb - j1 * 36.0
    j2 = fdiv(remb, 6.0)
    j3 = remb - j2 * 6.0
    in_b = (rf >= _OB) & (rf < _OC)
    m = m | (in_b & ((cf == 148.0 + j1) | (cf == 158.0 + j2)
                     | (cf == 164.0 + j3)))

    rc = rf - _OC
    k1 = fdiv(rc, 2.0)
    k2 = rc - k1 * 2.0
    in_c = (rf >= _OC) & (rf < _OC + 4)
    m = m | (in_c & ((cf == 170.0 + k1) | (cf == 172.0 + k2)))

    sel = m.astype(jnp.float32)
    t_ref[...] = jnp.dot(sel, f, preferred_element_type=jnp.float32)


def _prep_body(x_ref, wcomb_ref, add_ref, pk_ref):
    pk_ref[...] = (jnp.dot(x_ref[...], wcomb_ref[...],
                           preferred_element_type=jnp.float32)
                   + add_ref[...])


def _sc_body(t_hbm, pk_hbm, out_hbm, t_v, pk0_v, pk1_v, ao0_v, ao1_v, sems):
    wid = lax.axis_index("s") * 2 + lax.axis_index("c")
    row0 = wid * _RPW
    is_last = wid == _NW - 1
    nfull = jnp.where(is_last, 29, _NFULL)
    pltpu.sync_copy(t_hbm, t_v)
    bufs = [(pk0_v, ao0_v), (pk1_v, ao1_v)]

    def start_in(r0, b, rows):
        pltpu.async_copy(pk_hbm.at[pl.ds(r0, rows)],
                         bufs[b][0].at[pl.ds(0, rows)], sems.at[b])

    def wait_in(b, rows):
        pltpu.make_async_copy(pk_hbm.at[pl.ds(0, rows)],
                              bufs[b][0].at[pl.ds(0, rows)],
                              sems.at[b]).wait()

    def start_out(r0, b, rows):
        pltpu.async_copy(bufs[b][1].at[pl.ds(0, rows)],
                         out_hbm.at[pl.ds(r0, rows)], sems.at[2 + b])

    def wait_out(b, rows):
        pltpu.make_async_copy(bufs[b][1].at[pl.ds(0, rows)],
                              out_hbm.at[pl.ds(0, rows)],
                              sems.at[2 + b]).wait()

    def compute(b, rows):
        pk_b, ao_b = bufs[b]

        def row_body(r):
            ivec = pk_b[r, pl.ds(_EMB, 16)].astype(jnp.int32)
            i0 = ivec[0]
            i1 = ivec[1]
            i2 = ivec[2]
            i3 = ivec[3]
            for k in range(4):
                o = k * 16
                ao_b[r, pl.ds(o, 16)] = (
                    pk_b[r, pl.ds(o, 16)]
                    + t_v[pl.ds(i0 + o, 16)]
                    + t_v[pl.ds(i1 + o, 16)]
                    + t_v[pl.ds(i2 + o, 16)]
                    + t_v[pl.ds(i3 + o, 16)])

        plsc.parallel_loop(0, rows, 1, unroll=4)(row_body)

    start_in(row0, 0, _C)
    start_in(row0 + _C, 1, _C)

    def chunk_body(ci, carry):
        def do(b):
            wait_in(b, _C)

            @pl.when(ci >= 2)
            def _():
                wait_out(b, _C)

            compute(b, _C)
            r0 = row0 + ci * _C
            start_out(r0, b, _C)

            @pl.when(ci + 2 < nfull)
            def _():
                start_in(row0 + (ci + 2) * _C, b, _C)

        @pl.when(ci % 2 == 0)
        def _():
            do(0)

        @pl.when(ci % 2 == 1)
        def _():
            do(1)

        return carry

    lax.fori_loop(0, nfull, chunk_body, 0)
    wait_out(0, _C)
    wait_out(1, _C)

    @pl.when(jnp.logical_not(is_last))
    def _():
        r0 = row0 + _NFULL * _C
        start_in(r0, 0, _CT)
        wait_in(0, _CT)
        compute(0, _CT)
        start_out(r0, 0, _CT)
        wait_out(0, _CT)


def kernel(x, emb0, emb1, emb2, emb3, emb4, emb5, emb6, emb7, emb8, W, b):
    n = x.shape[0]
    w1 = W[:_EMB]

    t = pl.pallas_call(
        _build_body,
        out_shape=jax.ShapeDtypeStruct((_TROWS, _EMB), jnp.float32),
    )(emb0, emb1, emb2, emb3, emb4, emb5, emb6, emb7, emb8, w1)

    wc_np = np.zeros((_NF, _PK), np.float32)
    wc_np[0, _EMB + 0] = 64.0
    wc_np[1, _EMB + 1], wc_np[2, _EMB + 1], wc_np[3, _EMB + 1] = (
        144.0 * 64, 12.0 * 64, 64.0)
    wc_np[4, _EMB + 2], wc_np[5, _EMB + 2], wc_np[6, _EMB + 2] = (
        36.0 * 64, 6.0 * 64, 64.0)
    wc_np[7, _EMB + 3], wc_np[8, _EMB + 3] = 2.0 * 64, 64.0
    wcomb = jnp.asarray(wc_np).at[_NCAT:, :_EMB].set(W[_EMB:])
    add_np = np.zeros((1, _PK), np.float32)
    add_np[0, _EMB:_EMB + 4] = [0.5, _OA * 64 + 0.5, _OB * 64 + 0.5,
                                _OC * 64 + 0.5]
    add = jnp.asarray(add_np).at[0, :_EMB].set(b)

    nb = (n + _BLK - 1) // _BLK
    pk = pl.pallas_call(
        _prep_body,
        grid=(nb,),
        in_specs=[
            pl.BlockSpec((_BLK, _NF), lambda i: (i, 0)),
            pl.BlockSpec((_NF, _PK), lambda i: (0, 0)),
            pl.BlockSpec((1, _PK), lambda i: (0, 0)),
        ],
        out_specs=pl.BlockSpec((_BLK, _PK), lambda i: (i, 0)),
        out_shape=jax.ShapeDtypeStruct((n, _PK), jnp.float32),
    )(x, wcomb, add)

    sc_fn = functools.partial(
        pl.kernel,
        mesh=plsc.VectorSubcoreMesh(core_axis_name="c", subcore_axis_name="s"),
        out_type=jax.ShapeDtypeStruct((n, _EMB), jnp.float32),
        scratch_types=[
            pltpu.VMEM((_TROWS * _EMB,), jnp.float32),
            pltpu.VMEM((_C, _PK), jnp.float32),
            pltpu.VMEM((_C, _PK), jnp.float32),
            pltpu.VMEM((_C, _EMB), jnp.float32),
            pltpu.VMEM((_C, _EMB), jnp.float32),
            pltpu.SemaphoreType.DMA((4,)),
        ],
    )(_sc_body)
    return sc_fn(t.reshape(-1), pk)
